# baseline (device time: 18025 ns/iter reference)
import jax
import jax.numpy as jnp
from jax import lax
from jax.experimental import pallas as pl
from jax.experimental.pallas import tpu as pltpu

N_DEV = 4
SEGS = 2


def kernel(x, w_mat):
    m, k_per = x.shape
    _, n = w_mat.shape
    m_out = m // N_DEV
    half = n // 2
    seg = half // SEGS

    def body(x_ref, w_ref, out_ref, partial_ref, x_bf_ref, w_bf_ref,
             send_ref, recv_ref, send_sems, recv_sems):
        my = lax.axis_index("i")
        left = lax.rem(my + (N_DEV - 1), N_DEV)
        right = lax.rem(my + 1, N_DEV)

        barrier_sem = pltpu.get_barrier_semaphore()
        for nbr in (left, right):
            pl.semaphore_signal(
                barrier_sem, inc=1,
                device_id=(nbr,), device_id_type=pl.DeviceIdType.MESH,
            )

        x_bf_ref[...] = x_ref[...].astype(jnp.bfloat16)
        w_bf_ref[...] = w_ref[...].astype(jnp.bfloat16)

        def gemm_chunk(c):
            partial_ref[pl.ds(c * m_out, m_out), :] = jnp.dot(
                x_bf_ref[pl.ds(c * m_out, m_out), :],
                w_bf_ref[...],
                preferred_element_type=jnp.float32,
            )

        def rows(d, t):
            c = lax.rem(my + (N_DEV - 1 - t), N_DEV) if d == 0 else \
                lax.rem(my + 1 + t, N_DEV)
            return pl.ds(c * m_out, m_out)

        def pcols(d, s):
            return pl.ds(d * half + s * seg, seg)

        def bcols(s):
            return pl.ds(s * seg, seg)

        def make_rdma(d, t, s):
            return pltpu.make_async_remote_copy(
                src_ref=send_ref.at[d, t, :, bcols(s)],
                dst_ref=recv_ref.at[d, t, :, bcols(s)],
                send_sem=send_sems.at[d, t, s],
                recv_sem=recv_sems.at[d, t, s],
                device_id=(right if d == 0 else left,),
                device_id_type=pl.DeviceIdType.MESH,
            )

        rdmas = {}

        gemm_chunk(lax.rem(my + (N_DEV - 1), N_DEV))
        pl.semaphore_wait(barrier_sem, 2)
        for s in range(SEGS):
            send_ref[0, 0, :, bcols(s)] = (
                partial_ref[rows(0, 0), pcols(0, s)].astype(jnp.bfloat16)
            )
            r = rdmas[(0, 0, s)] = make_rdma(0, 0, s)
            r.start()
        gemm_chunk(lax.rem(my + 1, N_DEV))
        for s in range(SEGS):
            send_ref[1, 0, :, bcols(s)] = (
                partial_ref[rows(1, 0), pcols(1, s)].astype(jnp.bfloat16)
            )
            r = rdmas[(1, 0, s)] = make_rdma(1, 0, s)
            r.start()
        gemm_chunk(lax.rem(my + 2, N_DEV))
        gemm_chunk(my)

        for t in range(1, N_DEV - 1):
            for s in range(SEGS):
                for d in range(2):
                    rdmas[(d, t - 1, s)].wait_recv()
                    acc = (
                        recv_ref[d, t - 1, :, bcols(s)].astype(jnp.float32)
                        + partial_ref[rows(d, t), pcols(d, s)]
                    )
                    send_ref[d, t, :, bcols(s)] = acc.astype(jnp.bfloat16)
                    r = rdmas[(d, t, s)] = make_rdma(d, t, s)
                    r.start()

        for s in range(SEGS):
            for d in range(2):
                rdmas[(d, N_DEV - 2, s)].wait_recv()
                y = (
                    recv_ref[d, N_DEV - 2, :, bcols(s)].astype(jnp.float32)
                    + partial_ref[pl.ds(my * m_out, m_out), pcols(d, s)]
                )
                out_ref[:, pcols(d, s)] = y * jax.nn.sigmoid(y)

        for r in rdmas.values():
            r.wait_send()

    return pl.pallas_call(
        body,
        out_shape=jax.ShapeDtypeStruct((m_out, n), jnp.float32),
        in_specs=[
            pl.BlockSpec(memory_space=pltpu.VMEM),
            pl.BlockSpec(memory_space=pltpu.VMEM),
        ],
        out_specs=pl.BlockSpec(memory_space=pltpu.VMEM),
        scratch_shapes=[
            pltpu.VMEM((m, n), jnp.float32),
            pltpu.VMEM((m, k_per), jnp.bfloat16),
            pltpu.VMEM((k_per, n), jnp.bfloat16),
            pltpu.VMEM((2, N_DEV - 1, m_out, half), jnp.bfloat16),
            pltpu.VMEM((2, N_DEV - 1, m_out, half), jnp.bfloat16),
            pltpu.SemaphoreType.DMA((2, N_DEV - 1, SEGS)),
            pltpu.SemaphoreType.DMA((2, N_DEV - 1, SEGS)),
        ],
        compiler_params=pltpu.CompilerParams(collective_id=0),
    )(x, w_mat)


# device time: 16407 ns/iter; 1.0986x vs baseline; 1.0986x over previous
import jax
import jax.numpy as jnp
from jax import lax
from jax.experimental import pallas as pl
from jax.experimental.pallas import tpu as pltpu

N_DEV = 4

A1, A2, A3, B1, B2, B3 = range(6)


def kernel(x, w_mat):
    m, k_per = x.shape
    _, n = w_mat.shape
    m_out = m // N_DEV
    half = n // 2

    col_a = pl.ds(0, half)
    col_b = pl.ds(half, half)

    def body(x_ref, w_ref, out_ref, partial_ref, pbf_ref, x_bf, w_bf,
             stage_ref, recv_ref, send_sems, recv_sems):
        my = lax.axis_index("i")
        left = lax.rem(my + (N_DEV - 1), N_DEV)
        right = lax.rem(my + 1, N_DEV)

        barrier_sem = pltpu.get_barrier_semaphore()
        for nbr in (left, right):
            pl.semaphore_signal(
                barrier_sem, inc=1,
                device_id=(nbr,), device_id_type=pl.DeviceIdType.MESH,
            )

        x_bf[...] = x_ref[...].astype(jnp.bfloat16)
        w_bf[...] = w_ref[...].astype(jnp.bfloat16)

        def rows(c):
            return pl.ds(lax.rem(my + c, N_DEV) * m_out, m_out)

        def gemm_chunk(c):
            r = rows(c)
            p = jnp.dot(x_bf[r, :], w_bf[...],
                        preferred_element_type=jnp.float32)
            partial_ref[r, :] = p
            pbf_ref[r, :] = p.astype(jnp.bfloat16)

        def msg(i, src, tgt):
            return pltpu.make_async_remote_copy(
                src_ref=src,
                dst_ref=recv_ref.at[i],
                send_sem=send_sems.at[i],
                recv_sem=recv_sems.at[i],
                device_id=(tgt,),
                device_id_type=pl.DeviceIdType.MESH,
            )

        msgs = {}

        gemm_chunk(2)
        pl.semaphore_wait(barrier_sem, 2)
        msgs[A2] = msg(A2, pbf_ref.at[rows(2), col_a], left)
        msgs[A2].start()
        msgs[B2] = msg(B2, pbf_ref.at[rows(2), col_b], right)
        msgs[B2].start()

        gemm_chunk(1)
        msgs[A1] = msg(A1, pbf_ref.at[rows(1), col_a], right)
        msgs[A1].start()
        gemm_chunk(3)
        msgs[B1] = msg(B1, pbf_ref.at[rows(3), col_b], left)
        msgs[B1].start()
        gemm_chunk(0)

        msgs[A2].wait_recv()
        stage_ref[0, :, :] = (
            recv_ref[A2].astype(jnp.float32) + partial_ref[rows(3), col_a]
        ).astype(jnp.bfloat16)
        msgs[A3] = msg(A3, stage_ref.at[0], left)
        msgs[A3].start()

        msgs[B2].wait_recv()
        stage_ref[1, :, :] = (
            recv_ref[B2].astype(jnp.float32) + partial_ref[rows(1), col_b]
        ).astype(jnp.bfloat16)
        msgs[B3] = msg(B3, stage_ref.at[1], right)
        msgs[B3].start()

        msgs[A1].wait_recv()
        msgs[A3].wait_recv()
        y = (
            partial_ref[rows(0), col_a]
            + recv_ref[A1].astype(jnp.float32)
            + recv_ref[A3].astype(jnp.float32)
        )
        out_ref[:, col_a] = y * jax.nn.sigmoid(y)

        msgs[B1].wait_recv()
        msgs[B3].wait_recv()
        y = (
            partial_ref[rows(0), col_b]
            + recv_ref[B1].astype(jnp.float32)
            + recv_ref[B3].astype(jnp.float32)
        )
        out_ref[:, col_b] = y * jax.nn.sigmoid(y)

        for r in msgs.values():
            r.wait_send()

    return pl.pallas_call(
        body,
        out_shape=jax.ShapeDtypeStruct((m_out, n), jnp.float32),
        in_specs=[
            pl.BlockSpec(memory_space=pltpu.VMEM),
            pl.BlockSpec(memory_space=pltpu.VMEM),
        ],
        out_specs=pl.BlockSpec(memory_space=pltpu.VMEM),
        scratch_shapes=[
            pltpu.VMEM((m, n), jnp.float32),
            pltpu.VMEM((m, n), jnp.bfloat16),
            pltpu.VMEM((m, k_per), jnp.bfloat16),
            pltpu.VMEM((k_per, n), jnp.bfloat16),
            pltpu.VMEM((2, m_out, half), jnp.bfloat16),
            pltpu.VMEM((6, m_out, half), jnp.bfloat16),
            pltpu.SemaphoreType.DMA((6,)),
            pltpu.SemaphoreType.DMA((6,)),
        ],
        compiler_params=pltpu.CompilerParams(collective_id=0),
    )(x, w_mat)
